# baseline (speedup 1.0000x reference)
import functools

import jax
import jax.numpy as jnp
from jax import lax
from jax.experimental import pallas as pl
from jax.experimental.pallas import tpu as pltpu
from jax.experimental.pallas import tpu_sc as plsc

NC = 2
NS = 16
GROUP = 128
SUP = 8


def _matmul_half_layout(x, W):
    N, DIN = x.shape
    DOUT = W.shape[0]
    H = DOUT // NC
    BM = 1000

    def body(x_ref, w_ref, o_ref):
        o_ref[...] = lax.dot_general(
            x_ref[...], w_ref[...],
            dimension_numbers=(((1,), (1,)), ((), ())),
            preferred_element_type=jnp.float32)

    return pl.pallas_call(
        body,
        grid=(NC, N // BM),
        in_specs=[
            pl.BlockSpec((BM, DIN), lambda c, m: (m, 0)),
            pl.BlockSpec((H, DIN), lambda c, m: (c, 0)),
        ],
        out_specs=pl.BlockSpec((BM, H), lambda c, m: (c * (N // BM) + m, 0)),
        out_shape=jax.ShapeDtypeStruct((NC * N, H), jnp.float32),
    )(x, W)


def _sc_spmm(h2, b3, srcg, dstg, wg, N, H):
    EG = srcg.shape[0]
    GPT = EG // NS
    NSUP = GPT // SUP
    RB = 80
    NBLK = N // RB
    BPT = -(-NBLK // NS)

    mesh = plsc.VectorSubcoreMesh(
        core_axis_name="c", subcore_axis_name="s", num_cores=NC, num_subcores=NS)

    @functools.partial(
        pl.kernel,
        out_type=jax.ShapeDtypeStruct((N, NC * H), jnp.float32),
        mesh=mesh,
        scratch_types=[
            pltpu.VMEM((SUP, GROUP), jnp.int32),
            pltpu.VMEM((SUP, GROUP), jnp.int32),
            pltpu.VMEM((SUP, GROUP), jnp.float32),
            pltpu.VMEM((GROUP, H), jnp.float32),
            pltpu.VMEM((GROUP, H), jnp.float32),
            pltpu.VMEM((1, H), jnp.float32),
            pltpu.VMEM_SHARED((N, H), jnp.float32),
            pltpu.SemaphoreType.DMA,
            pltpu.SemaphoreType.DMA,
            pltpu.SemaphoreType.DMA,
            pltpu.SemaphoreType.DMA,
        ],
    )
    def spmm(h2_hbm, b3_hbm, srcg_hbm, dstg_hbm, wg_hbm, out_hbm,
             src_v, dst_v, w_v, buf0, buf1, b_v, acc,
             sem_g0, sem_g1, sem_s0, sem_s1):
        c = lax.axis_index("c")
        s = lax.axis_index("s")
        bufs = (buf0, buf1)
        gsems = (sem_g0, sem_g1)
        ssems = (sem_s0, sem_s1)

        pltpu.sync_copy(b3_hbm.at[c], b_v)

        def fill_row(r, _):
            for d in range(H // 16):
                sl = pl.ds(d * 16, 16)
                buf0[r, sl] = b_v[0, sl]
            return 0
        lax.fori_loop(0, RB, fill_row, 0)

        def init_blk(i, _):
            blk = s + i * NS

            @pl.when(blk < NBLK)
            def _():
                pltpu.sync_copy(buf0.at[pl.ds(0, RB)], acc.at[pl.ds(blk * RB, RB)])
            return 0
        lax.fori_loop(0, BPT, init_blk, 0)
        plsc.subcore_barrier()

        def scale_group(p, q):
            buf = bufs[p]

            @plsc.parallel_loop(0, GROUP // 16, 1, unroll=2)
            def body16(j):
                wv = w_v[q, pl.ds(j * 16, 16)]
                for t in range(16):
                    ws = wv[t]
                    row = j * 16 + t
                    for d in range(H // 16):
                        sl = pl.ds(d * 16, 16)
                        buf[row, sl] = buf[row, sl] * ws

        h2c = h2_hbm.at[pl.ds(pl.multiple_of(c * N, 8), N)]

        def fire_gather(p, q):
            return pltpu.async_copy(h2c.at[src_v.at[q]], bufs[p], gsems[p])

        def superchunk(ci, _):
            g0 = s * GPT + ci * SUP
            pltpu.sync_copy(srcg_hbm.at[pl.ds(g0, SUP)], src_v)
            pltpu.sync_copy(dstg_hbm.at[pl.ds(g0, SUP)], dst_v)
            pltpu.sync_copy(wg_hbm.at[pl.ds(g0, SUP)], w_v)
            gat = {0: None, 1: None}
            gat[0] = fire_gather(0, 0)
            scat = None
            for q in range(SUP):
                p = q % 2
                gat[p].wait()
                scale_group(p, q)
                if scat is not None:
                    scat.wait()
                if q + 1 < SUP:
                    gat[1 - p] = fire_gather(1 - p, q + 1)
                scat = pltpu.async_copy(bufs[p], acc.at[dst_v.at[q]],
                                        ssems[p], add=True)
            scat.wait()
            return 0
        lax.fori_loop(0, NSUP, superchunk, 0)
        plsc.subcore_barrier()

        def outblk(i, _):
            blk = s + i * NS

            @pl.when(blk < NBLK)
            def _():
                pltpu.sync_copy(acc.at[pl.ds(blk * RB, RB)], buf0.at[pl.ds(0, RB)])
                pltpu.sync_copy(buf0.at[pl.ds(0, RB)],
                                out_hbm.at[pl.ds(blk * RB, RB), pl.ds(c * H, H)])
            return 0
        lax.fori_loop(0, BPT, outblk, 0)

    return spmm(h2, b3, srcg, dstg, wg)


def kernel(input, edge_index, edge_weight, W, b):
    x = input
    N = x.shape[0]
    DOUT = W.shape[0]
    H = DOUT // NC

    dst = edge_index[0].astype(jnp.int32)
    src = edge_index[1].astype(jnp.int32)
    w = edge_weight.astype(jnp.float32)
    E = src.shape[0]

    EG = -(-E // (GROUP * NS * SUP)) * (NS * SUP)
    pad = EG * GROUP - E
    srcg = jnp.pad(src, (0, pad)).reshape(EG, GROUP)
    dstg = jnp.pad(dst, (0, pad)).reshape(EG, GROUP)
    wg = jnp.pad(w, (0, pad)).reshape(EG, GROUP)
    b3 = b.astype(jnp.float32).reshape(NC, 1, H)

    h2 = _matmul_half_layout(x, W)
    return _sc_spmm(h2, b3, srcg, dstg, wg, N, H)


# speedup vs baseline: 1.1210x; 1.1210x over previous
import functools

import jax
import jax.numpy as jnp
from jax import lax
from jax.experimental import pallas as pl
from jax.experimental.pallas import tpu as pltpu
from jax.experimental.pallas import tpu_sc as plsc

NC = 2
NS = 16
GROUP = 128
SUP = 16


def _matmul_half_layout(x, W):
    N, DIN = x.shape
    DOUT = W.shape[0]
    H = DOUT // NC
    BM = 1000

    def body(x_ref, w_ref, o_ref):
        o_ref[...] = lax.dot_general(
            x_ref[...], w_ref[...],
            dimension_numbers=(((1,), (1,)), ((), ())),
            preferred_element_type=jnp.float32)

    return pl.pallas_call(
        body,
        grid=(NC, N // BM),
        in_specs=[
            pl.BlockSpec((BM, DIN), lambda c, m: (m, 0)),
            pl.BlockSpec((H, DIN), lambda c, m: (c, 0)),
        ],
        out_specs=pl.BlockSpec((BM, H), lambda c, m: (c * (N // BM) + m, 0)),
        out_shape=jax.ShapeDtypeStruct((NC * N, H), jnp.float32),
    )(x, W)


def _sc_spmm(h2, b3, srcg, dstg, wg, N, H):
    EG = srcg.shape[0]
    GPT = EG // NS
    NSUP = GPT // SUP
    RB = 80
    NBLK = N // RB
    BPT = -(-NBLK // NS)

    mesh = plsc.VectorSubcoreMesh(
        core_axis_name="c", subcore_axis_name="s", num_cores=NC, num_subcores=NS)

    @functools.partial(
        pl.kernel,
        out_type=jax.ShapeDtypeStruct((N, NC * H), jnp.float32),
        mesh=mesh,
        scratch_types=[
            pltpu.VMEM((SUP, GROUP), jnp.int32),
            pltpu.VMEM((SUP, GROUP), jnp.int32),
            pltpu.VMEM((SUP, GROUP), jnp.float32),
            pltpu.VMEM((GROUP, H), jnp.float32),
            pltpu.VMEM((GROUP, H), jnp.float32),
            pltpu.VMEM((1, H), jnp.float32),
            pltpu.VMEM_SHARED((N, H), jnp.float32),
            pltpu.SemaphoreType.DMA,
            pltpu.SemaphoreType.DMA,
            pltpu.SemaphoreType.DMA,
            pltpu.SemaphoreType.DMA,
        ],
    )
    def spmm(h2_hbm, b3_hbm, srcg_hbm, dstg_hbm, wg_hbm, out_hbm,
             src_v, dst_v, w_v, buf0, buf1, b_v, acc,
             sem_g0, sem_g1, sem_s0, sem_s1):
        c = lax.axis_index("c")
        s = lax.axis_index("s")
        bufs = (buf0, buf1)
        gsems = (sem_g0, sem_g1)
        ssems = (sem_s0, sem_s1)

        pltpu.sync_copy(b3_hbm.at[c], b_v)

        def fill_row(r, _):
            for d in range(H // 16):
                sl = pl.ds(d * 16, 16)
                buf0[r, sl] = b_v[0, sl]
            return 0
        lax.fori_loop(0, RB, fill_row, 0)

        def init_blk(i, _):
            blk = s + i * NS

            @pl.when(blk < NBLK)
            def _():
                pltpu.sync_copy(buf0.at[pl.ds(0, RB)], acc.at[pl.ds(blk * RB, RB)])
            return 0
        lax.fori_loop(0, BPT, init_blk, 0)
        plsc.subcore_barrier()

        def scale_group(p, q):
            buf = bufs[p]

            @plsc.parallel_loop(0, GROUP // 16, 1, unroll=2)
            def body16(j):
                wv = w_v[q, pl.ds(j * 16, 16)]
                for t in range(16):
                    ws = wv[t]
                    row = j * 16 + t
                    for d in range(H // 16):
                        sl = pl.ds(d * 16, 16)
                        buf[row, sl] = buf[row, sl] * ws

        h2c = h2_hbm.at[pl.ds(pl.multiple_of(c * N, 8), N)]

        def fire_gather(p, q):
            return pltpu.async_copy(h2c.at[src_v.at[q]], bufs[p], gsems[p])

        def superchunk(ci, _):
            g0 = s * GPT + ci * SUP
            pltpu.sync_copy(srcg_hbm.at[pl.ds(g0, SUP)], src_v)
            pltpu.sync_copy(dstg_hbm.at[pl.ds(g0, SUP)], dst_v)
            pltpu.sync_copy(wg_hbm.at[pl.ds(g0, SUP)], w_v)
            gat = {0: None, 1: None}
            gat[0] = fire_gather(0, 0)
            for q in range(SUP):
                p = q % 2
                gat[p].wait()
                if q + 1 < SUP:
                    gat[1 - p] = fire_gather(1 - p, q + 1)
                scale_group(p, q)
                pltpu.sync_copy(bufs[p], acc.at[dst_v.at[q]], add=True)
            return 0
        lax.fori_loop(0, NSUP, superchunk, 0)
        plsc.subcore_barrier()

        def outblk(i, _):
            blk = s + i * NS

            @pl.when(blk < NBLK)
            def _():
                pltpu.sync_copy(acc.at[pl.ds(blk * RB, RB)], buf0.at[pl.ds(0, RB)])
                pltpu.sync_copy(buf0.at[pl.ds(0, RB)],
                                out_hbm.at[pl.ds(blk * RB, RB), pl.ds(c * H, H)])
            return 0
        lax.fori_loop(0, BPT, outblk, 0)

    return spmm(h2, b3, srcg, dstg, wg)


def kernel(input, edge_index, edge_weight, W, b):
    x = input
    N = x.shape[0]
    DOUT = W.shape[0]
    H = DOUT // NC

    dst = edge_index[0].astype(jnp.int32)
    src = edge_index[1].astype(jnp.int32)
    w = edge_weight.astype(jnp.float32)
    E = src.shape[0]

    EG = -(-E // (GROUP * NS * SUP)) * (NS * SUP)
    pad = EG * GROUP - E
    srcg = jnp.pad(src, (0, pad)).reshape(EG, GROUP)
    dstg = jnp.pad(dst, (0, pad)).reshape(EG, GROUP)
    wg = jnp.pad(w, (0, pad)).reshape(EG, GROUP)
    b3 = b.astype(jnp.float32).reshape(NC, 1, H)

    h2 = _matmul_half_layout(x, W)
    return _sc_spmm(h2, b3, srcg, dstg, wg, N, H)


# speedup vs baseline: 1.1225x; 1.0013x over previous
import functools

import jax
import jax.numpy as jnp
from jax import lax
from jax.experimental import pallas as pl
from jax.experimental.pallas import tpu as pltpu
from jax.experimental.pallas import tpu_sc as plsc

NC = 2
NS = 16
GROUP = 128
SUP = 16


def _matmul_half_layout(x, W):
    N, DIN = x.shape
    DOUT = W.shape[0]
    H = DOUT // NC
    BM = 1000

    def body(x_ref, w_ref, o_ref):
        o_ref[...] = lax.dot_general(
            x_ref[...], w_ref[...],
            dimension_numbers=(((1,), (1,)), ((), ())),
            preferred_element_type=jnp.float32)

    return pl.pallas_call(
        body,
        grid=(NC, N // BM),
        in_specs=[
            pl.BlockSpec((BM, DIN), lambda c, m: (m, 0)),
            pl.BlockSpec((H, DIN), lambda c, m: (c, 0)),
        ],
        out_specs=pl.BlockSpec((BM, H), lambda c, m: (c * (N // BM) + m, 0)),
        out_shape=jax.ShapeDtypeStruct((NC * N, H), jnp.float32),
    )(x, W)


def _sc_spmm(h2, b3, srcg, dstg, wg, N, H):
    EG = srcg.shape[0]
    GPT = EG // NS
    NSUP = GPT // SUP
    RB = 80
    NBLK = N // RB
    BPT = -(-NBLK // NS)

    mesh = plsc.VectorSubcoreMesh(
        core_axis_name="c", subcore_axis_name="s", num_cores=NC, num_subcores=NS)

    @functools.partial(
        pl.kernel,
        out_type=jax.ShapeDtypeStruct((N, NC * H), jnp.float32),
        mesh=mesh,
        scratch_types=[
            pltpu.VMEM((SUP, GROUP), jnp.int32),
            pltpu.VMEM((SUP, GROUP), jnp.int32),
            pltpu.VMEM((SUP, GROUP), jnp.float32),
            pltpu.VMEM((GROUP, H), jnp.float32),
            pltpu.VMEM((GROUP, H), jnp.float32),
            pltpu.VMEM((1, H), jnp.float32),
            pltpu.VMEM_SHARED((N, H), jnp.float32),
            pltpu.SemaphoreType.DMA,
            pltpu.SemaphoreType.DMA,
            pltpu.SemaphoreType.DMA,
            pltpu.SemaphoreType.DMA,
        ],
    )
    def spmm(h2_hbm, b3_hbm, srcg_hbm, dstg_hbm, wg_hbm, out_hbm,
             src_v, dst_v, w_v, buf0, buf1, b_v, acc,
             sem_g0, sem_g1, sem_s0, sem_s1):
        c = lax.axis_index("c")
        s = lax.axis_index("s")
        bufs = (buf0, buf1)
        gsems = (sem_g0, sem_g1)
        ssems = (sem_s0, sem_s1)

        pltpu.sync_copy(b3_hbm.at[c], b_v)

        def fill_row(r, _):
            for d in range(H // 16):
                sl = pl.ds(d * 16, 16)
                buf0[r, sl] = b_v[0, sl]
            return 0
        lax.fori_loop(0, RB, fill_row, 0)

        def init_blk(i, _):
            blk = s + i * NS

            @pl.when(blk < NBLK)
            def _():
                pltpu.sync_copy(buf0.at[pl.ds(0, RB)], acc.at[pl.ds(blk * RB, RB)])
            return 0
        lax.fori_loop(0, BPT, init_blk, 0)
        plsc.subcore_barrier()

        def scale_group(p, q):
            buf = bufs[p]

            @plsc.parallel_loop(0, GROUP // 16, 1, unroll=2)
            def body16(j):
                wv = w_v[q, pl.ds(j * 16, 16)]
                for t in range(16):
                    ws = wv[t]
                    row = j * 16 + t
                    for d in range(H // 16):
                        sl = pl.ds(d * 16, 16)
                        buf[row, sl] = buf[row, sl] * ws

        h2c = h2_hbm.at[pl.ds(pl.multiple_of(c * N, 8), N)]

        def fire_gather(p, q):
            return pltpu.async_copy(h2c.at[src_v.at[q]], bufs[p], gsems[p])

        def superchunk(ci, _):
            g0 = s * GPT + ci * SUP
            pltpu.sync_copy(srcg_hbm.at[pl.ds(g0, SUP)], src_v)
            pltpu.sync_copy(dstg_hbm.at[pl.ds(g0, SUP)], dst_v)
            pltpu.sync_copy(wg_hbm.at[pl.ds(g0, SUP)], w_v)
            gat = {0: None, 1: None}
            gat[0] = fire_gather(0, 0)
            for q in range(SUP):
                p = q % 2
                gat[p].wait()
                if q + 1 < SUP:
                    gat[1 - p] = fire_gather(1 - p, q + 1)
                scale_group(p, q)
                pltpu.sync_copy(bufs[p], acc.at[dst_v.at[q]], add=True)
            return 0
        lax.fori_loop(0, NSUP, superchunk, 0)
        plsc.subcore_barrier()

        def outblk(i, _):
            blk = s + i * NS

            @pl.when(blk < NBLK)
            def _():
                pltpu.sync_copy(acc.at[pl.ds(blk * RB, RB)],
                                out_hbm.at[pl.ds(blk * RB, RB), pl.ds(c * H, H)])
            return 0
        lax.fori_loop(0, BPT, outblk, 0)

    return spmm(h2, b3, srcg, dstg, wg)


def kernel(input, edge_index, edge_weight, W, b):
    x = input
    N = x.shape[0]
    DOUT = W.shape[0]
    H = DOUT // NC

    dst = edge_index[0].astype(jnp.int32)
    src = edge_index[1].astype(jnp.int32)
    w = edge_weight.astype(jnp.float32)
    E = src.shape[0]

    EG = -(-E // (GROUP * NS * SUP)) * (NS * SUP)
    pad = EG * GROUP - E
    srcg = jnp.pad(src, (0, pad)).reshape(EG, GROUP)
    dstg = jnp.pad(dst, (0, pad)).reshape(EG, GROUP)
    wg = jnp.pad(w, (0, pad)).reshape(EG, GROUP)
    b3 = b.astype(jnp.float32).reshape(NC, 1, H)

    h2 = _matmul_half_layout(x, W)
    return _sc_spmm(h2, b3, srcg, dstg, wg, N, H)
